# baseline (speedup 1.0000x reference)
import jax
import jax.numpy as jnp
from jax import lax
from jax.experimental import pallas as pl
from jax.experimental.pallas import tpu as pltpu
from jax.experimental.pallas import tpu_sc as plsc

B = 1024
H = 128
S = 100000
K = 8
ST = 2000
NT = S // ST
BS = 16
NBT = ST // BS
NCAND = K * BS
EPS = 1e-8
NEG = -1e30
IMAX = 2**31 - 1

SC_CORES = 2
SC_SUBCORES = 16
NW = SC_CORES * SC_SUBCORES
GCHUNK = 128


BIGF = 16384.0


def _sweep_cols(work_v, work_i, n_out):
    wif = work_i.astype(jnp.float32)
    wneg = BIGF - wif
    out_v, out_i = [], []
    v = work_v
    for _ in range(n_out):
        m = jnp.max(v, axis=0, keepdims=True)
        t = jnp.where(v == m, wneg, 0.0)
        amf = BIGF - jnp.max(t, axis=0, keepdims=True)
        out_v.append(m)
        out_i.append(amf.astype(jnp.int32))
        v = jnp.where(wif == amf, NEG, v)
    return jnp.concatenate(out_v, axis=0), jnp.concatenate(out_i, axis=0)


def _blocks_body(x_ref, a_ref, wa_ref, qn_ref, bm_ref, qn_s):
    i = pl.program_id(0)

    @pl.when(i == 0)
    def _init():
        q = lax.dot_general(x_ref[...], wa_ref[...],
                            (((1,), (1,)), ((), ())),
                            preferred_element_type=jnp.float32)
        qn2 = jnp.sum(q * q, axis=1, keepdims=True)
        qn = q * (1.0 / jnp.maximum(jnp.sqrt(qn2), EPS))
        qn_s[...] = qn
        qn_ref[...] = qn

    a = a_ref[...]
    n2 = jnp.sum(a * a, axis=1, keepdims=True)
    an = a * (1.0 / jnp.maximum(jnp.sqrt(n2), EPS))
    simsT = lax.dot_general(an, qn_s[...], (((1,), (1,)), ((), ())),
                            preferred_element_type=jnp.float32)
    bm_ref[...] = jnp.max(simsT.reshape(NBT, BS, B), axis=1)[None]


def _run_blocks(x, addresses, W_addr):
    return pl.pallas_call(
        _blocks_body,
        grid=(NT,),
        in_specs=[
            pl.BlockSpec((B, H), lambda i: (0, 0)),
            pl.BlockSpec((ST, H), lambda i: (i, 0)),
            pl.BlockSpec((H, H), lambda i: (0, 0)),
        ],
        out_specs=[
            pl.BlockSpec((B, H), lambda i: (0, 0)),
            pl.BlockSpec((1, NBT, B), lambda i: (i, 0, 0)),
        ],
        out_shape=[
            jax.ShapeDtypeStruct((B, H), jnp.float32),
            jax.ShapeDtypeStruct((NT, NBT, B), jnp.float32),
        ],
        scratch_shapes=[
            pltpu.VMEM((B, H), jnp.float32),
        ],
        compiler_params=pltpu.CompilerParams(
            dimension_semantics=("arbitrary",)),
    )(x, addresses, W_addr)


NBC = 5
BMROWS = S // BS // NBC


def _bsel_body(bm_ref, candT_ref, rv_s, ri_s):
    g = pl.program_id(0)

    @pl.when(g == 0)
    def _init():
        rv_s[...] = jnp.full((K, B), NEG, jnp.float32)
        ri_s[...] = jnp.zeros((K, B), jnp.int32)

    bm = bm_ref[0]
    work_v = jnp.concatenate([bm, rv_s[...]], axis=0)
    bids = g * BMROWS + lax.broadcasted_iota(jnp.int32, (BMROWS, B), 0)
    work_i = jnp.concatenate([bids, ri_s[...]], axis=0)
    new_v, new_i = _sweep_cols(work_v, work_i, K)
    rv_s[...] = new_v
    ri_s[...] = new_i

    @pl.when(g == NBC - 1)
    def _emit():
        ji = lax.broadcasted_iota(jnp.int32, (BS, B), 0)
        cols = [new_i[k:k + 1, :] * BS + ji for k in range(K)]
        candT_ref[...] = jnp.concatenate(cols, axis=0)


def _run_bsel(bm3):
    return pl.pallas_call(
        _bsel_body,
        grid=(NBC,),
        in_specs=[
            pl.BlockSpec((1, BMROWS, B), lambda g: (g, 0, 0)),
        ],
        out_specs=pl.BlockSpec((NCAND, B), lambda g: (0, 0)),
        out_shape=jax.ShapeDtypeStruct((NCAND, B), jnp.int32),
        scratch_shapes=[
            pltpu.VMEM((K, B), jnp.float32),
            pltpu.VMEM((K, B), jnp.int32),
        ],
        compiler_params=pltpu.CompilerParams(
            dimension_semantics=("arbitrary",)),
    )(bm3)


def _sc_gather_body(nchunk, idx_hbm, table_hbm, out_hbm, idx_v,
                    rows0_v, rows1_v, sem0, sem1):
    c = lax.axis_index("c")
    s = lax.axis_index("s")
    wid = s * SC_CORES + c

    pltpu.sync_copy(idx_hbm.at[pl.ds(wid * nchunk, nchunk)], idx_v)

    def start(j, rows_v, sem):
        return pltpu.async_copy(table_hbm.at[idx_v.at[j]], rows_v, sem)

    def drain(j, rows_v, sem):
        pltpu.make_async_copy(table_hbm.at[idx_v.at[j]], rows_v, sem).wait()
        pltpu.sync_copy(
            rows_v, out_hbm.at[pl.ds((wid * nchunk + j) * GCHUNK, GCHUNK)])

    start(0, rows0_v, sem0)

    def pair(g, _):
        j0 = 2 * g
        start(j0 + 1, rows1_v, sem1)
        drain(j0, rows0_v, sem0)

        @pl.when(j0 + 2 < nchunk)
        def _prefetch():
            start(j0 + 2, rows0_v, sem0)

        drain(j0 + 1, rows1_v, sem1)
        return _

    lax.fori_loop(0, nchunk // 2, pair, None)


def _gather_rows(idx2, table):
    n = idx2.shape[0] * GCHUNK
    nchunk = idx2.shape[0] // NW
    mesh = plsc.VectorSubcoreMesh(core_axis_name="c", subcore_axis_name="s",
                                  num_cores=SC_CORES,
                                  num_subcores=SC_SUBCORES)
    import functools
    run = pl.kernel(
        functools.partial(_sc_gather_body, nchunk),
        out_type=jax.ShapeDtypeStruct((n, H), jnp.float32),
        mesh=mesh,
        scratch_types=[
            pltpu.VMEM((nchunk, GCHUNK), jnp.int32),
            pltpu.VMEM((GCHUNK, H), jnp.float32),
            pltpu.VMEM((GCHUNK, H), jnp.float32),
            pltpu.SemaphoreType.DMA,
            pltpu.SemaphoreType.DMA,
        ],
    )
    return run(idx2, table)


def _csims_body(ac_ref, qn_ref, sims_ref):
    qn = qn_ref[...]
    ac = ac_ref[...]
    ones = jnp.full((H, 1), 1.0, jnp.float32)
    cols = []
    for c in range(8):
        a = ac[c]
        n2 = lax.dot_general(a * a, ones, (((1,), (0,)), ((), ())),
                             preferred_element_type=jnp.float32)
        d = lax.dot_general(a * qn, ones, (((1,), (0,)), ((), ())),
                            preferred_element_type=jnp.float32)
        cols.append(d * (1.0 / jnp.maximum(jnp.sqrt(n2), EPS)))
    sims_ref[...] = jnp.concatenate(cols, axis=1)[None]


def _run_csims(ac, qn):
    return pl.pallas_call(
        _csims_body,
        grid=(NCAND // 8,),
        in_specs=[
            pl.BlockSpec((8, B, H), lambda i: (i, 0, 0)),
            pl.BlockSpec((B, H), lambda i: (0, 0)),
        ],
        out_specs=pl.BlockSpec((1, B, 8), lambda i: (i, 0, 0)),
        out_shape=jax.ShapeDtypeStruct((NCAND // 8, B, 8), jnp.float32),
        compiler_params=pltpu.CompilerParams(
            dimension_semantics=("arbitrary",)),
    )(ac, qn)


def _sweep_rows(vals, ids, n_out):
    out_v, out_i = [], []
    v = vals
    for _ in range(n_out):
        m = jnp.max(v, axis=1, keepdims=True)
        cand = jnp.where(v == m, ids, IMAX)
        am = jnp.min(cand, axis=1, keepdims=True)
        out_v.append(m)
        out_i.append(am)
        v = jnp.where(cand == am, NEG, v)
    return jnp.concatenate(out_v, axis=1), jnp.concatenate(out_i, axis=1)


def _select_body(sims_ref, candQ_ref, vals_ref, idx_ref):
    vals, idx = _sweep_rows(sims_ref[...], candQ_ref[...], K)
    vals_ref[...] = vals
    idx_ref[...] = idx


def _run_select(sims, candQ):
    return pl.pallas_call(
        _select_body,
        in_specs=[
            pl.BlockSpec((B, NCAND), lambda: (0, 0)),
            pl.BlockSpec((B, NCAND), lambda: (0, 0)),
        ],
        out_specs=[
            pl.BlockSpec((B, K), lambda: (0, 0)),
            pl.BlockSpec((B, K), lambda: (0, 0)),
        ],
        out_shape=[
            jax.ShapeDtypeStruct((B, K), jnp.float32),
            jax.ShapeDtypeStruct((B, K), jnp.int32),
        ],
    )(sims, candQ)


def _combine_body(vals_ref, g_ref, wr_ref, out_ref):
    v = vals_ref[...]
    m = jnp.max(v, axis=1, keepdims=True)
    e = jnp.exp(v - m)
    w = e / jnp.sum(e, axis=1, keepdims=True)
    acc = w[:, 0:1] * g_ref[:, 0, :]
    for k in range(1, K):
        acc = acc + w[:, k:k + 1] * g_ref[:, k, :]
    out_ref[...] = lax.dot_general(acc, wr_ref[...],
                                   (((1,), (1,)), ((), ())),
                                   preferred_element_type=jnp.float32)


def _run_combine(vals, gathered, W_read):
    return pl.pallas_call(
        _combine_body,
        in_specs=[
            pl.BlockSpec((B, K), lambda: (0, 0)),
            pl.BlockSpec((B, K, H), lambda: (0, 0, 0)),
            pl.BlockSpec((H, H), lambda: (0, 0)),
        ],
        out_specs=pl.BlockSpec((B, H), lambda: (0, 0)),
        out_shape=jax.ShapeDtypeStruct((B, H), jnp.float32),
    )(vals, gathered, W_read)


def kernel(x, addresses, contents, W_addr, W_read):
    qn, bm3 = _run_blocks(x, addresses, W_addr)
    candT = _run_bsel(bm3.reshape(NBC, BMROWS, B))
    ac = _gather_rows(candT.reshape(-1, GCHUNK), addresses)
    sims3 = _run_csims(ac.reshape(NCAND, B, H), qn)
    sims = sims3.transpose(1, 0, 2).reshape(B, NCAND)
    vals, idx = _run_select(sims, candT.T)
    g = _gather_rows(idx.reshape(-1, GCHUNK), contents)
    return _run_combine(vals, g.reshape(B, K, H), W_read)


# speedup vs baseline: 1.0060x; 1.0060x over previous
import jax
import jax.numpy as jnp
from jax import lax
from jax.experimental import pallas as pl
from jax.experimental.pallas import tpu as pltpu
from jax.experimental.pallas import tpu_sc as plsc

B = 1024
H = 128
S = 100000
K = 8
ST = 4000
NT = S // ST
BS = 16
NBT = ST // BS
NCAND = K * BS
EPS = 1e-8
NEG = -1e30
IMAX = 2**31 - 1

SC_CORES = 2
SC_SUBCORES = 16
NW = SC_CORES * SC_SUBCORES
GCHUNK = 128


BIGF = 16384.0


def _sweep_cols(work_v, work_i, n_out):
    wif = work_i.astype(jnp.float32)
    wneg = BIGF - wif
    out_v, out_i = [], []
    v = work_v
    for _ in range(n_out):
        m = jnp.max(v, axis=0, keepdims=True)
        t = jnp.where(v == m, wneg, 0.0)
        amf = BIGF - jnp.max(t, axis=0, keepdims=True)
        out_v.append(m)
        out_i.append(amf.astype(jnp.int32))
        v = jnp.where(wif == amf, NEG, v)
    return jnp.concatenate(out_v, axis=0), jnp.concatenate(out_i, axis=0)


def _blocks_body(x_ref, a_ref, wa_ref, qn_ref, bm_ref, qn_s):
    i = pl.program_id(0)

    @pl.when(i == 0)
    def _init():
        q = lax.dot_general(x_ref[...], wa_ref[...],
                            (((1,), (1,)), ((), ())),
                            preferred_element_type=jnp.float32)
        qn2 = jnp.sum(q * q, axis=1, keepdims=True)
        qn = q * (1.0 / jnp.maximum(jnp.sqrt(qn2), EPS))
        qn_s[...] = qn
        qn_ref[...] = qn

    a = a_ref[...]
    n2 = jnp.sum(a * a, axis=1, keepdims=True)
    an = a * (1.0 / jnp.maximum(jnp.sqrt(n2), EPS))
    simsT = lax.dot_general(an, qn_s[...], (((1,), (1,)), ((), ())),
                            preferred_element_type=jnp.float32)
    bm_ref[...] = jnp.max(simsT.reshape(NBT, BS, B), axis=1)[None]


def _run_blocks(x, addresses, W_addr):
    return pl.pallas_call(
        _blocks_body,
        grid=(NT,),
        in_specs=[
            pl.BlockSpec((B, H), lambda i: (0, 0)),
            pl.BlockSpec((ST, H), lambda i: (i, 0)),
            pl.BlockSpec((H, H), lambda i: (0, 0)),
        ],
        out_specs=[
            pl.BlockSpec((B, H), lambda i: (0, 0)),
            pl.BlockSpec((1, NBT, B), lambda i: (i, 0, 0)),
        ],
        out_shape=[
            jax.ShapeDtypeStruct((B, H), jnp.float32),
            jax.ShapeDtypeStruct((NT, NBT, B), jnp.float32),
        ],
        scratch_shapes=[
            pltpu.VMEM((B, H), jnp.float32),
        ],
        compiler_params=pltpu.CompilerParams(
            dimension_semantics=("arbitrary",)),
    )(x, addresses, W_addr)


NBC = 2
BMROWS = S // BS // NBC


def _bsel_body(bm_ref, candT_ref, rv_s, ri_s):
    g = pl.program_id(0)

    @pl.when(g == 0)
    def _init():
        rv_s[...] = jnp.full((K, B), NEG, jnp.float32)
        ri_s[...] = jnp.zeros((K, B), jnp.int32)

    bm = bm_ref[0]
    work_v = jnp.concatenate([bm, rv_s[...]], axis=0)
    bids = g * BMROWS + lax.broadcasted_iota(jnp.int32, (BMROWS, B), 0)
    work_i = jnp.concatenate([bids, ri_s[...]], axis=0)
    new_v, new_i = _sweep_cols(work_v, work_i, K)
    rv_s[...] = new_v
    ri_s[...] = new_i

    @pl.when(g == NBC - 1)
    def _emit():
        ji = lax.broadcasted_iota(jnp.int32, (BS, B), 0)
        cols = [new_i[k:k + 1, :] * BS + ji for k in range(K)]
        candT_ref[...] = jnp.concatenate(cols, axis=0)


def _run_bsel(bm3):
    return pl.pallas_call(
        _bsel_body,
        grid=(NBC,),
        in_specs=[
            pl.BlockSpec((1, BMROWS, B), lambda g: (g, 0, 0)),
        ],
        out_specs=pl.BlockSpec((NCAND, B), lambda g: (0, 0)),
        out_shape=jax.ShapeDtypeStruct((NCAND, B), jnp.int32),
        scratch_shapes=[
            pltpu.VMEM((K, B), jnp.float32),
            pltpu.VMEM((K, B), jnp.int32),
        ],
        compiler_params=pltpu.CompilerParams(
            dimension_semantics=("arbitrary",)),
    )(bm3)


def _sc_gather_body(nchunk, idx_hbm, table_hbm, out_hbm, idx_v,
                    rows0_v, rows1_v, sem0, sem1):
    c = lax.axis_index("c")
    s = lax.axis_index("s")
    wid = s * SC_CORES + c

    pltpu.sync_copy(idx_hbm.at[pl.ds(wid * nchunk, nchunk)], idx_v)

    def start(j, rows_v, sem):
        return pltpu.async_copy(table_hbm.at[idx_v.at[j]], rows_v, sem)

    def drain(j, rows_v, sem):
        pltpu.make_async_copy(table_hbm.at[idx_v.at[j]], rows_v, sem).wait()
        pltpu.sync_copy(
            rows_v, out_hbm.at[pl.ds((wid * nchunk + j) * GCHUNK, GCHUNK)])

    start(0, rows0_v, sem0)

    def pair(g, _):
        j0 = 2 * g
        start(j0 + 1, rows1_v, sem1)
        drain(j0, rows0_v, sem0)

        @pl.when(j0 + 2 < nchunk)
        def _prefetch():
            start(j0 + 2, rows0_v, sem0)

        drain(j0 + 1, rows1_v, sem1)
        return _

    lax.fori_loop(0, nchunk // 2, pair, None)


def _gather_rows(idx2, table):
    n = idx2.shape[0] * GCHUNK
    nchunk = idx2.shape[0] // NW
    mesh = plsc.VectorSubcoreMesh(core_axis_name="c", subcore_axis_name="s",
                                  num_cores=SC_CORES,
                                  num_subcores=SC_SUBCORES)
    import functools
    run = pl.kernel(
        functools.partial(_sc_gather_body, nchunk),
        out_type=jax.ShapeDtypeStruct((n, H), jnp.float32),
        mesh=mesh,
        scratch_types=[
            pltpu.VMEM((nchunk, GCHUNK), jnp.int32),
            pltpu.VMEM((GCHUNK, H), jnp.float32),
            pltpu.VMEM((GCHUNK, H), jnp.float32),
            pltpu.SemaphoreType.DMA,
            pltpu.SemaphoreType.DMA,
        ],
    )
    return run(idx2, table)


def _csims_body(ac_ref, qn_ref, sims_ref):
    qn = qn_ref[...]
    ac = ac_ref[...]
    ones = jnp.full((H, 1), 1.0, jnp.float32)
    cols = []
    for c in range(16):
        a = ac[c]
        n2 = lax.dot_general(a * a, ones, (((1,), (0,)), ((), ())),
                             preferred_element_type=jnp.float32)
        d = lax.dot_general(a * qn, ones, (((1,), (0,)), ((), ())),
                            preferred_element_type=jnp.float32)
        cols.append(d * (1.0 / jnp.maximum(jnp.sqrt(n2), EPS)))
    sims_ref[...] = jnp.concatenate(cols, axis=1)[None]


def _run_csims(ac, qn):
    return pl.pallas_call(
        _csims_body,
        grid=(NCAND // 16,),
        in_specs=[
            pl.BlockSpec((16, B, H), lambda i: (i, 0, 0)),
            pl.BlockSpec((B, H), lambda i: (0, 0)),
        ],
        out_specs=pl.BlockSpec((1, B, 16), lambda i: (i, 0, 0)),
        out_shape=jax.ShapeDtypeStruct((NCAND // 16, B, 16), jnp.float32),
        compiler_params=pltpu.CompilerParams(
            dimension_semantics=("arbitrary",)),
    )(ac, qn)


def _sweep_rows(vals, ids, n_out):
    out_v, out_i = [], []
    v = vals
    for _ in range(n_out):
        m = jnp.max(v, axis=1, keepdims=True)
        cand = jnp.where(v == m, ids, IMAX)
        am = jnp.min(cand, axis=1, keepdims=True)
        out_v.append(m)
        out_i.append(am)
        v = jnp.where(cand == am, NEG, v)
    return jnp.concatenate(out_v, axis=1), jnp.concatenate(out_i, axis=1)


def _select_body(sims_ref, candQ_ref, vals_ref, idx_ref):
    vals, idx = _sweep_rows(sims_ref[...], candQ_ref[...], K)
    vals_ref[...] = vals
    idx_ref[...] = idx


def _run_select(sims, candQ):
    return pl.pallas_call(
        _select_body,
        in_specs=[
            pl.BlockSpec((B, NCAND), lambda: (0, 0)),
            pl.BlockSpec((B, NCAND), lambda: (0, 0)),
        ],
        out_specs=[
            pl.BlockSpec((B, K), lambda: (0, 0)),
            pl.BlockSpec((B, K), lambda: (0, 0)),
        ],
        out_shape=[
            jax.ShapeDtypeStruct((B, K), jnp.float32),
            jax.ShapeDtypeStruct((B, K), jnp.int32),
        ],
    )(sims, candQ)


def _combine_body(vals_ref, g_ref, wr_ref, out_ref):
    v = vals_ref[...]
    m = jnp.max(v, axis=1, keepdims=True)
    e = jnp.exp(v - m)
    w = e / jnp.sum(e, axis=1, keepdims=True)
    acc = w[:, 0:1] * g_ref[:, 0, :]
    for k in range(1, K):
        acc = acc + w[:, k:k + 1] * g_ref[:, k, :]
    out_ref[...] = lax.dot_general(acc, wr_ref[...],
                                   (((1,), (1,)), ((), ())),
                                   preferred_element_type=jnp.float32)


def _run_combine(vals, gathered, W_read):
    return pl.pallas_call(
        _combine_body,
        in_specs=[
            pl.BlockSpec((B, K), lambda: (0, 0)),
            pl.BlockSpec((B, K, H), lambda: (0, 0, 0)),
            pl.BlockSpec((H, H), lambda: (0, 0)),
        ],
        out_specs=pl.BlockSpec((B, H), lambda: (0, 0)),
        out_shape=jax.ShapeDtypeStruct((B, H), jnp.float32),
    )(vals, gathered, W_read)


def kernel(x, addresses, contents, W_addr, W_read):
    qn, bm3 = _run_blocks(x, addresses, W_addr)
    candT = _run_bsel(bm3.reshape(NBC, BMROWS, B))
    ac = _gather_rows(candT.reshape(-1, GCHUNK), addresses)
    sims3 = _run_csims(ac.reshape(NCAND, B, H), qn)
    sims = sims3.transpose(1, 0, 2).reshape(B, NCAND)
    vals, idx = _run_select(sims, candT.T)
    g = _gather_rows(idx.reshape(-1, GCHUNK), contents)
    return _run_combine(vals, g.reshape(B, K, H), W_read)


# speedup vs baseline: 1.0124x; 1.0063x over previous
import jax
import jax.numpy as jnp
from jax import lax
from jax.experimental import pallas as pl
from jax.experimental.pallas import tpu as pltpu
from jax.experimental.pallas import tpu_sc as plsc

B = 1024
H = 128
S = 100000
K = 8
ST = 4000
NT = S // ST
BS = 16
NBT = ST // BS
NCAND = K * BS
EPS = 1e-8
NEG = -1e30
IMAX = 2**31 - 1

SC_CORES = 2
SC_SUBCORES = 16
NW = SC_CORES * SC_SUBCORES
GCHUNK = 128


BIGF = 16384.0


def _sweep_cols(work_v, work_i, n_out):
    wif = work_i.astype(jnp.float32)
    wneg = BIGF - wif
    out_v, out_i = [], []
    v = work_v
    for _ in range(n_out):
        m = jnp.max(v, axis=0, keepdims=True)
        t = jnp.where(v == m, wneg, 0.0)
        amf = BIGF - jnp.max(t, axis=0, keepdims=True)
        out_v.append(m)
        out_i.append(amf.astype(jnp.int32))
        v = jnp.where(wif == amf, NEG, v)
    return jnp.concatenate(out_v, axis=0), jnp.concatenate(out_i, axis=0)


def _blocks_body(x_ref, a_ref, wa_ref, qn_ref, bm_ref, qn_s):
    i = pl.program_id(0)

    @pl.when(i == 0)
    def _init():
        q = lax.dot_general(x_ref[...], wa_ref[...],
                            (((1,), (1,)), ((), ())),
                            preferred_element_type=jnp.float32)
        qn2 = jnp.sum(q * q, axis=1, keepdims=True)
        qn = q * (1.0 / jnp.maximum(jnp.sqrt(qn2), EPS))
        qn_s[...] = qn
        qn_ref[...] = qn

    a = a_ref[...]
    n2 = jnp.sum(a * a, axis=1, keepdims=True)
    an = a * (1.0 / jnp.maximum(jnp.sqrt(n2), EPS))
    simsT = lax.dot_general(an, qn_s[...], (((1,), (1,)), ((), ())),
                            preferred_element_type=jnp.float32)
    bm_ref[...] = jnp.max(simsT.reshape(NBT, BS, B), axis=1)[None]


def _run_blocks(x, addresses, W_addr):
    return pl.pallas_call(
        _blocks_body,
        grid=(NT,),
        in_specs=[
            pl.BlockSpec((B, H), lambda i: (0, 0)),
            pl.BlockSpec((ST, H), lambda i: (i, 0)),
            pl.BlockSpec((H, H), lambda i: (0, 0)),
        ],
        out_specs=[
            pl.BlockSpec((B, H), lambda i: (0, 0)),
            pl.BlockSpec((1, NBT, B), lambda i: (i, 0, 0)),
        ],
        out_shape=[
            jax.ShapeDtypeStruct((B, H), jnp.float32),
            jax.ShapeDtypeStruct((NT, NBT, B), jnp.float32),
        ],
        scratch_shapes=[
            pltpu.VMEM((B, H), jnp.float32),
        ],
        compiler_params=pltpu.CompilerParams(
            dimension_semantics=("arbitrary",)),
    )(x, addresses, W_addr)


NBC = 2
BMROWS = S // BS // NBC


def _bsel_body(bm_ref, candT_ref, rv_s, ri_s):
    g = pl.program_id(0)

    @pl.when(g == 0)
    def _init():
        rv_s[...] = jnp.full((K, B), NEG, jnp.float32)
        ri_s[...] = jnp.zeros((K, B), jnp.int32)

    bm = bm_ref[0]
    work_v = jnp.concatenate([bm, rv_s[...]], axis=0)
    bids = g * BMROWS + lax.broadcasted_iota(jnp.int32, (BMROWS, B), 0)
    work_i = jnp.concatenate([bids, ri_s[...]], axis=0)
    new_v, new_i = _sweep_cols(work_v, work_i, K)
    rv_s[...] = new_v
    ri_s[...] = new_i

    @pl.when(g == NBC - 1)
    def _emit():
        ji = lax.broadcasted_iota(jnp.int32, (BS, B), 0)
        cols = [new_i[k:k + 1, :] * BS + ji for k in range(K)]
        candT_ref[...] = jnp.concatenate(cols, axis=0)


def _run_bsel(bm3):
    return pl.pallas_call(
        _bsel_body,
        grid=(NBC,),
        in_specs=[
            pl.BlockSpec((1, BMROWS, B), lambda g: (g, 0, 0)),
        ],
        out_specs=pl.BlockSpec((NCAND, B), lambda g: (0, 0)),
        out_shape=jax.ShapeDtypeStruct((NCAND, B), jnp.int32),
        scratch_shapes=[
            pltpu.VMEM((K, B), jnp.float32),
            pltpu.VMEM((K, B), jnp.int32),
        ],
        compiler_params=pltpu.CompilerParams(
            dimension_semantics=("arbitrary",)),
    )(bm3)


NBUF = 4


def _sc_gather_body(nchunk, idx_hbm, table_hbm, out_hbm, idx_v,
                    r0, r1, r2, r3, s0, s1, s2, s3):
    rows_vs = (r0, r1, r2, r3)
    sems = (s0, s1, s2, s3)
    c = lax.axis_index("c")
    s = lax.axis_index("s")
    wid = s * SC_CORES + c

    pltpu.sync_copy(idx_hbm.at[pl.ds(wid * nchunk, nchunk)], idx_v)

    def start(j, t):
        pltpu.async_copy(table_hbm.at[idx_v.at[j]], rows_vs[t], sems[t])

    def drain(j, t):
        pltpu.make_async_copy(
            table_hbm.at[idx_v.at[j]], rows_vs[t], sems[t]).wait()
        pltpu.sync_copy(
            rows_vs[t],
            out_hbm.at[pl.ds((wid * nchunk + j) * GCHUNK, GCHUNK)])

    depth = min(NBUF, nchunk)
    for t in range(depth):
        start(t, t)

    def group(g, _):
        j0 = NBUF * g
        for t in range(NBUF):
            drain(j0 + t, t)

            @pl.when(j0 + t + NBUF < nchunk)
            def _prefetch():
                start(j0 + t + NBUF, t)
        return _

    ngroups = nchunk // NBUF
    if ngroups:
        lax.fori_loop(0, ngroups, group, None)
    for j in range(ngroups * NBUF, nchunk):
        drain(j, j % NBUF)


def _gather_rows(idx2, table):
    n = idx2.shape[0] * GCHUNK
    nchunk = idx2.shape[0] // NW
    mesh = plsc.VectorSubcoreMesh(core_axis_name="c", subcore_axis_name="s",
                                  num_cores=SC_CORES,
                                  num_subcores=SC_SUBCORES)
    import functools
    run = pl.kernel(
        functools.partial(_sc_gather_body, nchunk),
        out_type=jax.ShapeDtypeStruct((n, H), jnp.float32),
        mesh=mesh,
        scratch_types=(
            [pltpu.VMEM((nchunk, GCHUNK), jnp.int32)]
            + [pltpu.VMEM((GCHUNK, H), jnp.float32)] * NBUF
            + [pltpu.SemaphoreType.DMA] * NBUF
        ),
    )
    return run(idx2, table)


def _csims_body(ac_ref, qn_ref, sims_ref):
    qn = qn_ref[...]
    ac = ac_ref[...]
    ones = jnp.full((H, 1), 1.0, jnp.float32)
    cols = []
    for c in range(16):
        a = ac[c]
        n2 = lax.dot_general(a * a, ones, (((1,), (0,)), ((), ())),
                             preferred_element_type=jnp.float32)
        d = lax.dot_general(a * qn, ones, (((1,), (0,)), ((), ())),
                            preferred_element_type=jnp.float32)
        cols.append(d * (1.0 / jnp.maximum(jnp.sqrt(n2), EPS)))
    sims_ref[...] = jnp.concatenate(cols, axis=1)[None]


def _run_csims(ac, qn):
    return pl.pallas_call(
        _csims_body,
        grid=(NCAND // 16,),
        in_specs=[
            pl.BlockSpec((16, B, H), lambda i: (i, 0, 0)),
            pl.BlockSpec((B, H), lambda i: (0, 0)),
        ],
        out_specs=pl.BlockSpec((1, B, 16), lambda i: (i, 0, 0)),
        out_shape=jax.ShapeDtypeStruct((NCAND // 16, B, 16), jnp.float32),
        compiler_params=pltpu.CompilerParams(
            dimension_semantics=("arbitrary",)),
    )(ac, qn)


def _sweep_rows(vals, ids, n_out):
    out_v, out_i = [], []
    v = vals
    for _ in range(n_out):
        m = jnp.max(v, axis=1, keepdims=True)
        cand = jnp.where(v == m, ids, IMAX)
        am = jnp.min(cand, axis=1, keepdims=True)
        out_v.append(m)
        out_i.append(am)
        v = jnp.where(cand == am, NEG, v)
    return jnp.concatenate(out_v, axis=1), jnp.concatenate(out_i, axis=1)


def _select_body(sims_ref, candQ_ref, vals_ref, idx_ref):
    vals, idx = _sweep_rows(sims_ref[...], candQ_ref[...], K)
    vals_ref[...] = vals
    idx_ref[...] = idx


def _run_select(sims, candQ):
    return pl.pallas_call(
        _select_body,
        in_specs=[
            pl.BlockSpec((B, NCAND), lambda: (0, 0)),
            pl.BlockSpec((B, NCAND), lambda: (0, 0)),
        ],
        out_specs=[
            pl.BlockSpec((B, K), lambda: (0, 0)),
            pl.BlockSpec((B, K), lambda: (0, 0)),
        ],
        out_shape=[
            jax.ShapeDtypeStruct((B, K), jnp.float32),
            jax.ShapeDtypeStruct((B, K), jnp.int32),
        ],
    )(sims, candQ)


def _combine_body(vals_ref, g_ref, wr_ref, out_ref):
    v = vals_ref[...]
    m = jnp.max(v, axis=1, keepdims=True)
    e = jnp.exp(v - m)
    w = e / jnp.sum(e, axis=1, keepdims=True)
    acc = w[:, 0:1] * g_ref[:, 0, :]
    for k in range(1, K):
        acc = acc + w[:, k:k + 1] * g_ref[:, k, :]
    out_ref[...] = lax.dot_general(acc, wr_ref[...],
                                   (((1,), (1,)), ((), ())),
                                   preferred_element_type=jnp.float32)


def _run_combine(vals, gathered, W_read):
    return pl.pallas_call(
        _combine_body,
        in_specs=[
            pl.BlockSpec((B, K), lambda: (0, 0)),
            pl.BlockSpec((B, K, H), lambda: (0, 0, 0)),
            pl.BlockSpec((H, H), lambda: (0, 0)),
        ],
        out_specs=pl.BlockSpec((B, H), lambda: (0, 0)),
        out_shape=jax.ShapeDtypeStruct((B, H), jnp.float32),
    )(vals, gathered, W_read)


def kernel(x, addresses, contents, W_addr, W_read):
    qn, bm3 = _run_blocks(x, addresses, W_addr)
    candT = _run_bsel(bm3.reshape(NBC, BMROWS, B))
    ac = _gather_rows(candT.reshape(-1, GCHUNK), addresses)
    sims3 = _run_csims(ac.reshape(NCAND, B, H), qn)
    sims = sims3.transpose(1, 0, 2).reshape(B, NCAND)
    vals, idx = _run_select(sims, candT.T)
    g = _gather_rows(idx.reshape(-1, GCHUNK), contents)
    return _run_combine(vals, g.reshape(B, K, H), W_read)
